# Initial kernel scaffold; baseline (speedup 1.0000x reference)
import jax
import jax.numpy as jnp
from jax.experimental import pallas as pl


def kernel(x, edge_index, W1l, W1r, att1, b1, W2l, W2r, att2, b2, W3l, W3r, att3, b3):
    raise NotImplementedError("write your pallas kernel here")


# speedup vs baseline: 25.1261x; 25.1261x over previous
import functools

import jax
import jax.numpy as jnp
import numpy as np
from jax import lax
from jax.experimental import pallas as pl
from jax.experimental.pallas import tpu as pltpu
from jax.experimental.pallas import tpu_sc as plsc

NC = 2
NS = 16
NW = NC * NS
K = 128
D = 128


@functools.lru_cache(maxsize=None)
def _make_sc_gather(n, nch):
  ep = NW * nch * K
  mesh = plsc.VectorSubcoreMesh(core_axis_name="c", subcore_axis_name="s")

  def body(xl_hbm, xr_hbm, src_hbm, dst_hbm, gj_hbm, gi_hbm,
           bs, bd, bj, bi, sem1, sem2):
    c = lax.axis_index("c")
    s = lax.axis_index("s")
    wid = s * NC + c

    def chunk(k, _):
      pltpu.sync_copy(src_hbm.at[wid, k], bs)
      pltpu.sync_copy(dst_hbm.at[wid, k], bd)
      cp1 = pltpu.async_copy(xl_hbm.at[bs], bj, sem1)
      cp2 = pltpu.async_copy(xr_hbm.at[bd], bi, sem2)
      cp1.wait()
      cp2.wait()
      r0 = (wid * nch + k) * K
      pltpu.sync_copy(bj, gj_hbm.at[pl.ds(r0, K)])
      pltpu.sync_copy(bi, gi_hbm.at[pl.ds(r0, K)])
      return 0

    lax.fori_loop(0, nch, chunk, 0)

  return pl.kernel(
      body,
      out_type=(jax.ShapeDtypeStruct((ep, D), jnp.float32),
                jax.ShapeDtypeStruct((ep, D), jnp.float32)),
      mesh=mesh,
      scratch_types=(
          pltpu.MemorySpace.VMEM((K,), jnp.int32),
          pltpu.MemorySpace.VMEM((K,), jnp.int32),
          pltpu.MemorySpace.VMEM((K, D), jnp.float32),
          pltpu.MemorySpace.VMEM((K, D), jnp.float32),
          pltpu.SemaphoreType.DMA,
          pltpu.SemaphoreType.DMA,
      ),
  )


@functools.lru_cache(maxsize=None)
def _make_sc_scatter(n, npad, nch):
  nchf = NW * nch
  cps = nchf // NS
  nps = npad // NS
  nzc = nps // K
  mesh = plsc.VectorSubcoreMesh(core_axis_name="c", subcore_axis_name="s")

  def body(yx_hbm, dst_hbm, z_hbm, out, spm, bidx, bv):
    c = lax.axis_index("c")
    s = lax.axis_index("s")
    r = s * nps

    pltpu.sync_copy(z_hbm, bv)

    def zchunk(j, _):
      pltpu.sync_copy(bv, spm.at[pl.ds(r + j * K, K)])
      return 0

    lax.fori_loop(0, nzc, zchunk, 0)
    plsc.subcore_barrier()

    def chunk(k, _):
      pltpu.sync_copy(dst_hbm.at[k], bidx)
      pltpu.sync_copy(yx_hbm.at[c, pl.ds(k * K, K)], bv)
      pltpu.sync_copy(bv, spm.at[bidx], add=True)
      return 0

    lax.fori_loop(s * cps, (s + 1) * cps, chunk, 0)
    plsc.subcore_barrier()

    def ochunk(j, _):
      pltpu.sync_copy(spm.at[pl.ds(r + j * K, K)], bv)
      pltpu.sync_copy(bv, out.at[c, pl.ds(r + j * K, K)])
      return 0

    lax.fori_loop(0, nzc, ochunk, 0)

  return pl.kernel(
      body,
      out_type=jax.ShapeDtypeStruct((NC, npad, D), jnp.float32),
      mesh=mesh,
      scratch_types=(
          pltpu.MemorySpace.VMEM_SHARED((npad, D), jnp.float32),
          pltpu.MemorySpace.VMEM((K,), jnp.int32),
          pltpu.MemorySpace.VMEM((K, D), jnp.float32),
      ),
  )


@functools.lru_cache(maxsize=None)
def _make_tc_first(n):
  R = 1000
  grid = n // R

  def body(x_ref, wl_ref, wr_ref, xl_ref, xr_ref):
    xb = x_ref[...]
    xl_ref[...] = jnp.dot(xb, wl_ref[...], preferred_element_type=jnp.float32)
    xr_ref[...] = jnp.dot(xb, wr_ref[...], preferred_element_type=jnp.float32)

  return pl.pallas_call(
      body,
      grid=(grid,),
      in_specs=[
          pl.BlockSpec((R, D), lambda i: (i, 0)),
          pl.BlockSpec((D, D), lambda i: (0, 0)),
          pl.BlockSpec((D, D), lambda i: (0, 0)),
      ],
      out_specs=[
          pl.BlockSpec((R, D), lambda i: (i, 0)),
          pl.BlockSpec((R, D), lambda i: (i, 0)),
      ],
      out_shape=[jax.ShapeDtypeStruct((n, D), jnp.float32),
                 jax.ShapeDtypeStruct((n, D), jnp.float32)],
  )


@functools.lru_cache(maxsize=None)
def _make_tc_edge(ep):
  RE = NW * K
  grid = ep // RE

  def body(gj_ref, gi_ref, att_ref, s_ref, yx_ref):
    gj = gj_ref[...]
    gi = gi_ref[...]
    u = gi + gj
    lr = jnp.maximum(u, 0.2 * u)
    v = lr * att_ref[...]
    ab = jnp.dot(v, s_ref[...], preferred_element_type=jnp.float32)
    exb = jnp.exp(ab)
    yx_ref[...] = jnp.stack([gj * exb, exb])

  return pl.pallas_call(
      body,
      grid=(grid,),
      in_specs=[
          pl.BlockSpec((RE, D), lambda i: (i, 0)),
          pl.BlockSpec((RE, D), lambda i: (i, 0)),
          pl.BlockSpec((1, D), lambda i: (0, 0)),
          pl.BlockSpec((D, D), lambda i: (0, 0)),
      ],
      out_specs=pl.BlockSpec((NC, RE, D), lambda i: (0, i, 0)),
      out_shape=jax.ShapeDtypeStruct((NC, ep, D), jnp.float32),
  )


@functools.lru_cache(maxsize=None)
def _make_tc_mid(n, relu, residual):
  R = 1000
  grid = n // R

  def body(*refs):
    if residual:
      (nd_ref, b_ref, res_ref, wl_ref, wr_ref,
       h_ref, xl_ref, xr_ref) = refs
    else:
      (nd_ref, b_ref, wl_ref, wr_ref,
       h_ref, xl_ref, xr_ref) = refs
    h = nd_ref[0] / (nd_ref[1] + 1e-16) + b_ref[...]
    if relu:
      h = jnp.maximum(h, 0.0)
    if residual:
      h = h + res_ref[...]
    h_ref[...] = h
    xl_ref[...] = jnp.dot(h, wl_ref[...], preferred_element_type=jnp.float32)
    xr_ref[...] = jnp.dot(h, wr_ref[...], preferred_element_type=jnp.float32)

  in_specs = [
      pl.BlockSpec((NC, R, D), lambda i: (0, i, 0)),
      pl.BlockSpec((1, D), lambda i: (0, 0)),
  ]
  if residual:
    in_specs.append(pl.BlockSpec((R, D), lambda i: (i, 0)))
  in_specs += [
      pl.BlockSpec((D, D), lambda i: (0, 0)),
      pl.BlockSpec((D, D), lambda i: (0, 0)),
  ]

  return pl.pallas_call(
      body,
      grid=(grid,),
      in_specs=in_specs,
      out_specs=[
          pl.BlockSpec((R, D), lambda i: (i, 0)),
          pl.BlockSpec((R, D), lambda i: (i, 0)),
          pl.BlockSpec((R, D), lambda i: (i, 0)),
      ],
      out_shape=[jax.ShapeDtypeStruct((n, D), jnp.float32)] * 3,
  )


@functools.lru_cache(maxsize=None)
def _make_tc_final(n):
  R = 1000
  grid = n // R

  def body(nd_ref, b_ref, out_ref):
    out_ref[...] = nd_ref[0] / (nd_ref[1] + 1e-16) + b_ref[...]

  return pl.pallas_call(
      body,
      grid=(grid,),
      in_specs=[
          pl.BlockSpec((NC, R, D), lambda i: (0, i, 0)),
          pl.BlockSpec((1, D), lambda i: (0, 0)),
      ],
      out_specs=pl.BlockSpec((R, D), lambda i: (i, 0)),
      out_shape=jax.ShapeDtypeStruct((n, D), jnp.float32),
  )


def _head_mat(heads):
  ch = D // heads
  return jnp.asarray(np.kron(np.eye(heads), np.ones((ch, ch))),
                     dtype=jnp.float32)


@jax.jit
def kernel(x, edge_index, W1l, W1r, att1, b1, W2l, W2r, att2, b2,
           W3l, W3r, att3, b3):
  n = x.shape[0]
  e = edge_index.shape[1]
  npad = -(-(n + 1) // (NS * K)) * (NS * K)
  nch = -(-e // (NW * K))
  ep = NW * nch * K
  pad = ep - e

  src = jnp.pad(edge_index[0], (0, pad)).reshape(NW, nch, K)
  dstg = jnp.pad(edge_index[1], (0, pad)).reshape(NW, nch, K)
  dsts = jnp.pad(edge_index[1], (0, pad),
                 constant_values=n).reshape(NW * nch, K)

  zrows = jnp.zeros((K, D), jnp.float32)
  s8 = _head_mat(8)
  s1 = _head_mat(1)

  gather = _make_sc_gather(n, nch)
  scatter = _make_sc_scatter(n, npad, nch)
  edge = _make_tc_edge(ep)

  def sc_scatter(yx):
    return scatter(yx, dsts, zrows)[:, :n]

  xl1, xr1 = _make_tc_first(n)(x, W1l, W1r)
  gj1, gi1 = gather(xl1, xr1, src, dstg)
  nd1 = sc_scatter(edge(gj1, gi1, att1.reshape(1, D), s8))

  h1, xl2, xr2 = _make_tc_mid(n, True, False)(
      nd1, b1.reshape(1, D), W2l, W2r)
  gj2, gi2 = gather(xl2, xr2, src, dstg)
  nd2 = sc_scatter(edge(gj2, gi2, att2.reshape(1, D), s8))

  _, xl3, xr3 = _make_tc_mid(n, False, True)(
      nd2, b2.reshape(1, D), h1, W3l, W3r)
  gj3, gi3 = gather(xl3, xr3, src, dstg)
  nd3 = sc_scatter(edge(gj3, gi3, att3.reshape(1, D), s1))

  return _make_tc_final(n)(nd3, b3.reshape(1, D))
